# Initial kernel scaffold; baseline (speedup 1.0000x reference)
import jax
import jax.numpy as jnp
from jax.experimental import pallas as pl


def kernel(x, table):
    raise NotImplementedError("write your pallas kernel here")


# speedup vs baseline: 3.0759x; 3.0759x over previous
import functools

import jax
import jax.numpy as jnp
from jax import lax
from jax.experimental import pallas as pl
from jax.experimental.pallas import tpu as pltpu
from jax.experimental.pallas import tpu_sc as plsc

_C = 128


@functools.lru_cache(maxsize=None)
def _build(B, D):
    info = plsc.get_sparse_core_info()
    nc, ns = info.num_cores, info.num_subcores
    nw = nc * ns
    b_per_w = B // nw
    n_chunks = b_per_w // _C
    mesh = plsc.VectorSubcoreMesh(core_axis_name="c", subcore_axis_name="s")

    def body(x_hbm, tab_hbm, out_hbm, idx_v, rows_v, sem):
        wid = lax.axis_index("s") * nc + lax.axis_index("c")
        base = wid * b_per_w
        pltpu.sync_copy(x_hbm.at[pl.ds(base, b_per_w)], idx_v)

        def chunk(c, carry):
            off = pl.multiple_of(c * _C, _C)
            pltpu.async_copy(
                tab_hbm.at[idx_v.at[pl.ds(off, _C)]], rows_v, sem
            ).wait()
            pltpu.sync_copy(rows_v, out_hbm.at[pl.ds(base + off, _C)])
            return carry

        lax.fori_loop(0, n_chunks, chunk, 0)

    return pl.kernel(
        body,
        mesh=mesh,
        out_type=jax.ShapeDtypeStruct((B, D), jnp.float32),
        scratch_types=[
            pltpu.VMEM((b_per_w,), jnp.int32),
            pltpu.VMEM((_C, D), jnp.float32),
            pltpu.SemaphoreType.DMA,
        ],
    )


def kernel(x, table):
    bt, h = x.shape
    _, d = table.shape
    b = bt * h
    idx = x.reshape(b).astype(jnp.int32)
    out = _build(b, d)(idx, table)
    return out.reshape(bt, h, d)


# speedup vs baseline: 3.4681x; 1.1275x over previous
import functools

import jax
import jax.numpy as jnp
from jax import lax
from jax.experimental import pallas as pl
from jax.experimental.pallas import tpu as pltpu
from jax.experimental.pallas import tpu_sc as plsc

_C = 128
_NBUF = 5


@functools.lru_cache(maxsize=None)
def _build(B, D):
    info = plsc.get_sparse_core_info()
    nc, ns = info.num_cores, info.num_subcores
    nw = nc * ns
    b_per_w = B // nw
    n_chunks = b_per_w // _C
    n_groups = n_chunks // _NBUF
    mesh = plsc.VectorSubcoreMesh(core_axis_name="c", subcore_axis_name="s")

    def body(x_hbm, tab_hbm, out_hbm, idx_v, rows_v, *sems):
        gsems, ssems = sems[:_NBUF], sems[_NBUF:]
        wid = lax.axis_index("s") * nc + lax.axis_index("c")
        base = wid * b_per_w
        pltpu.sync_copy(x_hbm.at[pl.ds(base, b_per_w)], idx_v)

        def start_gather(c, b):
            off = pl.multiple_of(c * _C, _C)
            pltpu.make_async_copy(
                tab_hbm.at[idx_v.at[pl.ds(off, _C)]], rows_v.at[b], gsems[b]
            ).start()

        def wait_gather(b):
            pltpu.make_async_copy(
                tab_hbm.at[pl.ds(0, _C)], rows_v.at[b], gsems[b]
            ).wait()

        def start_store(c, b):
            off = pl.multiple_of(c * _C, _C)
            pltpu.make_async_copy(
                rows_v.at[b], out_hbm.at[pl.ds(base + off, _C)], ssems[b]
            ).start()

        def wait_store(b):
            pltpu.make_async_copy(
                rows_v.at[b], out_hbm.at[pl.ds(0, _C)], ssems[b]
            ).wait()

        for b in range(_NBUF):
            start_gather(b, b)

        def group(g, carry):
            for b in range(_NBUF):
                c = g * _NBUF + b
                wait_gather(b)
                start_store(c, b)

                @pl.when(g < n_groups - 1)
                def _():
                    wait_store(b)
                    start_gather(c + _NBUF, b)

            return carry

        lax.fori_loop(0, n_groups, group, 0)
        for b in range(_NBUF):
            wait_store(b)

    return pl.kernel(
        body,
        mesh=mesh,
        out_type=jax.ShapeDtypeStruct((B, D), jnp.float32),
        scratch_types=[
            pltpu.VMEM((b_per_w,), jnp.int32),
            pltpu.VMEM((_NBUF, _C, D), jnp.float32),
        ]
        + [pltpu.SemaphoreType.DMA] * (2 * _NBUF),
    )


def kernel(x, table):
    bt, h = x.shape
    _, d = table.shape
    b = bt * h
    idx = x.reshape(b).astype(jnp.int32)
    out = _build(b, d)(idx, table)
    return out.reshape(bt, h, d)


# speedup vs baseline: 6.1887x; 1.7845x over previous
import functools

import jax
import jax.numpy as jnp
from jax import lax
from jax.experimental import pallas as pl
from jax.experimental.pallas import tpu as pltpu
from jax.experimental.pallas import tpu_sc as plsc

_NBUF = 8


@functools.lru_cache(maxsize=None)
def _build(bt, h, d):
    info = plsc.get_sparse_core_info()
    nc, ns = info.num_cores, info.num_subcores
    nw = nc * ns
    nb = bt // nw
    n_groups = nb // _NBUF
    mesh = plsc.VectorSubcoreMesh(core_axis_name="c", subcore_axis_name="s")

    def body(x_hbm, tab_hbm, out_hbm, idx_v, rows_v, *sems):
        gsems, ssems = sems[:_NBUF], sems[_NBUF:]
        wid = lax.axis_index("s") * nc + lax.axis_index("c")
        base = wid * nb
        pltpu.sync_copy(x_hbm.at[pl.ds(base, nb), :], idx_v)

        def start_gather(i, b):
            pltpu.make_async_copy(
                tab_hbm.at[idx_v.at[i]], rows_v.at[b], gsems[b]
            ).start()

        def wait_gather(b):
            pltpu.make_async_copy(
                tab_hbm.at[idx_v.at[0]], rows_v.at[b], gsems[b]
            ).wait()

        def start_store(i, b):
            pltpu.make_async_copy(
                rows_v.at[b], out_hbm.at[base + i], ssems[b]
            ).start()

        def wait_store(b):
            pltpu.make_async_copy(
                rows_v.at[b], out_hbm.at[0], ssems[b]
            ).wait()

        for b in range(_NBUF):
            start_gather(b, b)

        def group(g, carry):
            for b in range(_NBUF):
                i = g * _NBUF + b
                wait_gather(b)
                start_store(i, b)

                @pl.when(g < n_groups - 1)
                def _():
                    wait_store(b)
                    start_gather(i + _NBUF, b)

            return carry

        lax.fori_loop(0, n_groups, group, 0)
        for b in range(_NBUF):
            wait_store(b)

    return pl.kernel(
        body,
        mesh=mesh,
        out_type=jax.ShapeDtypeStruct((bt, h, d), jnp.float32),
        scratch_types=[
            pltpu.VMEM((nb, h), jnp.int32),
            pltpu.VMEM((_NBUF, h, d), jnp.float32),
        ]
        + [pltpu.SemaphoreType.DMA] * (2 * _NBUF),
        compiler_params=pltpu.CompilerParams(use_tc_tiling_on_sc=True),
    )


def kernel(x, table):
    bt, h = x.shape
    _, d = table.shape
    return _build(bt, h, d)(x.astype(jnp.int32), table)


# speedup vs baseline: 11.0904x; 1.7921x over previous
import functools

import jax
import jax.numpy as jnp
from jax import lax
from jax.experimental import pallas as pl
from jax.experimental.pallas import tpu as pltpu
from jax.experimental.pallas import tpu_sc as plsc

_C = 128
_NBUF = 5


@functools.lru_cache(maxsize=None)
def _build(bt, h, d):
    info = plsc.get_sparse_core_info()
    nc, ns = info.num_cores, info.num_subcores
    nw = nc * ns
    assert bt % (nw * _C) == 0
    n_groups = h // _NBUF
    assert h == n_groups * _NBUF
    mesh = plsc.VectorSubcoreMesh(core_axis_name="c", subcore_axis_name="s")

    def body(xt_hbm, tab_hbm, out_hbm, idx_v, rows_v, *sems):
        gsems, ssems = sems[:_NBUF], sems[_NBUF:]
        wid = lax.axis_index("s") * nc + lax.axis_index("c")
        base = wid * _C
        pltpu.sync_copy(xt_hbm.at[:, pl.ds(base, _C)], idx_v)

        def start_gather(j, b):
            pltpu.make_async_copy(
                tab_hbm.at[idx_v.at[j]], rows_v.at[b], gsems[b]
            ).start()

        def wait_gather(b):
            pltpu.make_async_copy(
                tab_hbm.at[idx_v.at[0]], rows_v.at[b], gsems[b]
            ).wait()

        def start_store(j, b):
            pltpu.make_async_copy(
                rows_v.at[b], out_hbm.at[j, pl.ds(base, _C)], ssems[b]
            ).start()

        def wait_store(b):
            pltpu.make_async_copy(
                rows_v.at[b], out_hbm.at[0, pl.ds(base, _C)], ssems[b]
            ).wait()

        for b in range(_NBUF):
            start_gather(b, b)

        def group(g, carry):
            for b in range(_NBUF):
                j = g * _NBUF + b
                wait_gather(b)
                start_store(j, b)

                @pl.when(g < n_groups - 1)
                def _():
                    wait_store(b)
                    start_gather(j + _NBUF, b)

            return carry

        lax.fori_loop(0, n_groups, group, 0)
        for b in range(_NBUF):
            wait_store(b)

    return pl.kernel(
        body,
        mesh=mesh,
        out_type=jax.ShapeDtypeStruct((h, bt, d), jnp.float32),
        scratch_types=[
            pltpu.VMEM((h, _C), jnp.int32),
            pltpu.VMEM((_NBUF, _C, d), jnp.float32),
        ]
        + [pltpu.SemaphoreType.DMA] * (2 * _NBUF),
        compiler_params=pltpu.CompilerParams(use_tc_tiling_on_sc=True),
    )


def kernel(x, table):
    bt, h = x.shape
    _, d = table.shape
    xt = jnp.transpose(x.astype(jnp.int32))
    out = _build(bt, h, d)(xt, table)
    return jnp.transpose(out, (1, 0, 2))


# speedup vs baseline: 11.0933x; 1.0003x over previous
import functools

import jax
import jax.numpy as jnp
from jax import lax
from jax.experimental import pallas as pl
from jax.experimental.pallas import tpu as pltpu
from jax.experimental.pallas import tpu_sc as plsc

_C = 128
_CH = 64
_NBUF = 10


@functools.lru_cache(maxsize=None)
def _build(bt, h, d):
    info = plsc.get_sparse_core_info()
    nc, ns = info.num_cores, info.num_subcores
    nw = nc * ns
    assert bt % (nw * _C) == 0
    n_chunks = h * (_C // _CH)
    n_groups = n_chunks // _NBUF
    assert n_chunks == n_groups * _NBUF
    mesh = plsc.VectorSubcoreMesh(core_axis_name="c", subcore_axis_name="s")

    def body(xt_hbm, tab_hbm, out_hbm, idx_v, rows_v, *sems):
        gsems, ssems = sems[:_NBUF], sems[_NBUF:]
        wid = lax.axis_index("s") * nc + lax.axis_index("c")
        base = wid * _C
        pltpu.sync_copy(xt_hbm.at[:, pl.ds(base, _C)], idx_v)

        def start_gather(c, b):
            j, half = c // 2, c % 2
            pltpu.make_async_copy(
                tab_hbm.at[idx_v.at[j, pl.ds(half * _CH, _CH)]],
                rows_v.at[b],
                gsems[b],
            ).start()

        def wait_gather(b):
            pltpu.make_async_copy(
                tab_hbm.at[idx_v.at[0, pl.ds(0, _CH)]], rows_v.at[b], gsems[b]
            ).wait()

        def start_store(c, b):
            j, half = c // 2, c % 2
            pltpu.make_async_copy(
                rows_v.at[b],
                out_hbm.at[j, pl.ds(base + half * _CH, _CH)],
                ssems[b],
            ).start()

        def wait_store(b):
            pltpu.make_async_copy(
                rows_v.at[b], out_hbm.at[0, pl.ds(base, _CH)], ssems[b]
            ).wait()

        for b in range(_NBUF):
            start_gather(b, b)

        def group(g, carry):
            for b in range(_NBUF):
                c = g * _NBUF + b
                wait_gather(b)
                start_store(c, b)

                @pl.when(g < n_groups - 1)
                def _():
                    wait_store(b)
                    start_gather(c + _NBUF, b)

            return carry

        lax.fori_loop(0, n_groups, group, 0)
        for b in range(_NBUF):
            wait_store(b)

    return pl.kernel(
        body,
        mesh=mesh,
        out_type=jax.ShapeDtypeStruct((h, bt, d), jnp.float32),
        scratch_types=[
            pltpu.VMEM((h, _C), jnp.int32),
            pltpu.VMEM((_NBUF, _CH, d), jnp.float32),
        ]
        + [pltpu.SemaphoreType.DMA] * (2 * _NBUF),
        compiler_params=pltpu.CompilerParams(use_tc_tiling_on_sc=True),
    )


def kernel(x, table):
    bt, h = x.shape
    _, d = table.shape
    xt = jnp.transpose(x.astype(jnp.int32))
    out = _build(bt, h, d)(xt, table)
    return jnp.transpose(out, (1, 0, 2))


# speedup vs baseline: 11.1485x; 1.0050x over previous
import functools

import jax
import jax.numpy as jnp
from jax import lax
from jax.experimental import pallas as pl
from jax.experimental.pallas import tpu as pltpu
from jax.experimental.pallas import tpu_sc as plsc

_C = 128
_NBUF = 5


@functools.lru_cache(maxsize=None)
def _build(bt, h, d):
    info = plsc.get_sparse_core_info()
    nc, ns = info.num_cores, info.num_subcores
    nw = nc * ns
    assert bt % (nw * _C) == 0
    n_groups = h // _NBUF
    assert h == n_groups * _NBUF
    mesh = plsc.VectorSubcoreMesh(core_axis_name="c", subcore_axis_name="s")

    def body(xt_hbm, tab_hbm, out_hbm, idx_v, rows_v, *sems):
        gsems, ssems = sems[:_NBUF], sems[_NBUF:]
        wid = lax.axis_index("s") * nc + lax.axis_index("c")
        base = wid * _C
        pltpu.sync_copy(xt_hbm.at[:, pl.ds(base, _C)], idx_v)

        def start_gather(j, b):
            pltpu.make_async_copy(
                tab_hbm.at[idx_v.at[j]], rows_v.at[b], gsems[b]
            ).start()

        def wait_gather(b):
            pltpu.make_async_copy(
                tab_hbm.at[idx_v.at[0]], rows_v.at[b], gsems[b]
            ).wait()

        def start_store(j, b):
            pltpu.make_async_copy(
                rows_v.at[b], out_hbm.at[j, pl.ds(base, _C)], ssems[b]
            ).start()

        def wait_store(b):
            pltpu.make_async_copy(
                rows_v.at[b], out_hbm.at[0, pl.ds(base, _C)], ssems[b]
            ).wait()

        for b in range(_NBUF):
            start_gather(b, b)

        def group(g, carry):
            for b in range(_NBUF):
                j = g * _NBUF + b
                wait_gather(b)
                start_store(j, b)

                @pl.when(g < n_groups - 1)
                def _():
                    wait_store(b)
                    start_gather(j + _NBUF, b)

            return carry

        lax.fori_loop(0, n_groups, group, 0)
        for b in range(_NBUF):
            wait_store(b)

    return pl.kernel(
        body,
        mesh=mesh,
        out_type=jax.ShapeDtypeStruct((h, bt, d), jnp.float32),
        scratch_types=[
            pltpu.VMEM((h, _C), jnp.int32),
            pltpu.VMEM((_NBUF, _C, d), jnp.float32),
        ]
        + [pltpu.SemaphoreType.DMA] * (2 * _NBUF),
        compiler_params=pltpu.CompilerParams(use_tc_tiling_on_sc=True),
    )


def kernel(x, table):
    bt, h = x.shape
    _, d = table.shape
    xt = jnp.transpose(x.astype(jnp.int32))
    out = _build(bt, h, d)(xt, table)
    return jnp.transpose(out, (1, 0, 2))


# speedup vs baseline: 11.1642x; 1.0014x over previous
import functools

import jax
import jax.numpy as jnp
from jax import lax
from jax.experimental import pallas as pl
from jax.experimental.pallas import tpu as pltpu
from jax.experimental.pallas import tpu_sc as plsc

_C = 128
_NBUF = 5


@functools.lru_cache(maxsize=None)
def _build(bt, h, d):
    info = plsc.get_sparse_core_info()
    nc, ns = info.num_cores, info.num_subcores
    nw = nc * ns
    assert bt % (nw * _C) == 0
    n_groups = h // _NBUF
    assert h == n_groups * _NBUF
    mesh = plsc.VectorSubcoreMesh(core_axis_name="c", subcore_axis_name="s")

    def body(xt_hbm, tab_hbm, out_hbm, idx_v, rows_v, *sems):
        gsems, ssems = sems[:_NBUF], sems[_NBUF:]
        wid = lax.axis_index("s") * nc + lax.axis_index("c")
        base = wid * _C
        pltpu.sync_copy(xt_hbm.at[:, pl.ds(base, _C)], idx_v)

        def start_gather(j, b):
            pltpu.make_async_copy(
                tab_hbm.at[idx_v.at[j]], rows_v.at[b], gsems[b]
            ).start()

        def wait_gather(b):
            pltpu.make_async_copy(
                tab_hbm.at[idx_v.at[0]], rows_v.at[b], gsems[b]
            ).wait()

        def start_store(j, b):
            pltpu.make_async_copy(
                rows_v.at[b], out_hbm.at[j, pl.ds(base, _C)], ssems[b]
            ).start()

        def wait_store(b):
            pltpu.make_async_copy(
                rows_v.at[b], out_hbm.at[0, pl.ds(base, _C)], ssems[b]
            ).wait()

        for b in range(_NBUF):
            start_gather(b, b)

        def group(g, carry):
            for b in range(_NBUF):
                j = g * _NBUF + b
                wait_gather(b)
                start_store(j, b)

                @pl.when(g < n_groups - 1)
                def _():
                    wait_store(b)
                    start_gather(j + _NBUF, b)

            return carry

        lax.fori_loop(0, n_groups, group, 0)
        for b in range(_NBUF):
            wait_store(b)

    return pl.kernel(
        body,
        mesh=mesh,
        out_type=jax.ShapeDtypeStruct((h, bt, d), jnp.float32),
        scratch_types=[
            pltpu.VMEM((h, _C), jnp.int32),
            pltpu.VMEM((_NBUF, _C, d), jnp.float32),
        ]
        + [pltpu.SemaphoreType.DMA] * (2 * _NBUF),
        compiler_params=pltpu.CompilerParams(
            use_tc_tiling_on_sc=True,
            disable_bounds_checks=True,
            disable_semaphore_checks=True,
        ),
    )


def kernel(x, table):
    bt, h = x.shape
    _, d = table.shape
    xt = jnp.transpose(x.astype(jnp.int32))
    out = _build(bt, h, d)(xt, table)
    return jnp.transpose(out, (1, 0, 2))
